# Initial kernel scaffold; baseline (speedup 1.0000x reference)
import jax
import jax.numpy as jnp
from jax.experimental import pallas as pl


def kernel(inputs, embeddings, ema_cluster_hidden, ema_dw_hidden, is_training):
    raise NotImplementedError("write your pallas kernel here")


# speedup vs baseline: 1.2727x; 1.2727x over previous
import functools

import jax
import jax.numpy as jnp
from jax import lax
from jax.experimental import pallas as pl
from jax.experimental.pallas import tpu as pltpu
from jax.experimental.pallas import tpu_sc as plsc

D = 256
K = 8192
N = 8192
BN = 1024
BK = 1024
NB = N // BN
KB = K // BK
DECAY = 0.99
EPS = 1e-05
COMMIT = 0.25

NC = 2
NS = 16
NW = NC * NS
GR = N // NW
SR = N // NS
DH = D // NC


def _argmin_body(x_ref, e_ref, idx_ref, min_s, arg_s):
    kb = pl.program_id(1)
    x = x_ref[...]
    e = e_ref[...]
    mm = jnp.dot(x, e, preferred_element_type=jnp.float32)
    x2 = jnp.sum(x * x, axis=1, keepdims=True)
    e2 = jnp.sum(e * e, axis=0, keepdims=True)
    d = x2 - 2.0 * mm + e2
    lmin = jnp.min(d, axis=1, keepdims=True)
    cols = lax.broadcasted_iota(jnp.int32, (BN, BK), 1) + kb * BK
    larg = jnp.min(jnp.where(d == lmin, cols, jnp.int32(K)), axis=1,
                   keepdims=True)

    @pl.when(kb == 0)
    def _():
        min_s[...] = lmin
        arg_s[...] = larg

    @pl.when(kb > 0)
    def _():
        upd = lmin < min_s[...]
        arg_s[...] = jnp.where(upd, larg, arg_s[...])
        min_s[...] = jnp.where(upd, lmin, min_s[...])

    @pl.when(kb == KB - 1)
    def _():
        idx_ref[...] = arg_s[...]


def _argmin_call(x, e):
    return pl.pallas_call(
        _argmin_body,
        grid=(NB, KB),
        in_specs=[
            pl.BlockSpec((BN, D), lambda n, k: (n, 0)),
            pl.BlockSpec((D, BK), lambda n, k: (0, k)),
        ],
        out_specs=pl.BlockSpec((BN, 1), lambda n, k: (n, 0)),
        out_shape=jax.ShapeDtypeStruct((N, 1), jnp.int32),
        scratch_shapes=[
            pltpu.VMEM((BN, 1), jnp.float32),
            pltpu.VMEM((BN, 1), jnp.int32),
        ],
    )(x, e)


@functools.cache
def _sc_kernel():
    mesh = plsc.VectorSubcoreMesh(core_axis_name="c", subcore_axis_name="s",
                                  num_cores=NC, num_subcores=NS)
    return functools.partial(
        pl.kernel,
        out_type=(
            jax.ShapeDtypeStruct((N, D), jnp.float32),
            jax.ShapeDtypeStruct((NC, K, DH), jnp.float32),
            jax.ShapeDtypeStruct((K,), jnp.float32),
        ),
        mesh=mesh,
        scratch_types=[
            pltpu.VMEM((GR // 128, 128), jnp.int32),
            pltpu.VMEM((128, D), jnp.float32),
            pltpu.VMEM((SR // 128, 128), jnp.int32),
            pltpu.VMEM((128, DH), jnp.float32),
            pltpu.VMEM((128,), jnp.float32),
            pltpu.VMEM_SHARED((K, DH), jnp.float32),
            pltpu.VMEM_SHARED((K,), jnp.float32),
            pltpu.SemaphoreType.DMA,
        ],
        compiler_params=pltpu.CompilerParams(use_tc_tiling_on_sc=False),
    )(_sc_body)


def _sc_body(et_hbm, x_hbm, idx_hbm, zeros_hbm, zeros1_hbm, ones1_hbm,
             quant_hbm, dwt_hbm, counts_hbm,
             gidx_v, grow_v, sidx_v, xbuf_v, ones_v, dw_acc, cnt_acc, sem):
    c = lax.axis_index("c")
    s = lax.axis_index("s")
    wid = s * NC + c
    rbase = s * SR

    pltpu.sync_copy(zeros_hbm, dw_acc.at[pl.ds(rbase, SR)])

    @pl.when(c == 0)
    def _():
        pltpu.sync_copy(zeros1_hbm, cnt_acc.at[pl.ds(rbase, SR)])
        pltpu.sync_copy(ones1_hbm, ones_v)

    gbase = wid * GR
    pltpu.sync_copy(idx_hbm.at[pl.ds(gbase // 128, GR // 128)], gidx_v)
    for j in range(GR // 128):
        pltpu.async_copy(et_hbm.at[gidx_v.at[j]], grow_v, sem).wait()
        pltpu.sync_copy(grow_v, quant_hbm.at[pl.ds(gbase + j * 128, 128)])

    pltpu.sync_copy(idx_hbm.at[pl.ds(s * (SR // 128), SR // 128)], sidx_v)

    plsc.subcore_barrier()

    for j in range(SR // 128):
        pltpu.sync_copy(x_hbm.at[pl.ds(rbase + j * 128, 128),
                                 pl.ds(c * DH, DH)], xbuf_v)
        pltpu.sync_copy(xbuf_v, dw_acc.at[sidx_v.at[j]], add=True)

    @pl.when(c == 0)
    def _():
        for j in range(SR // 128):
            pltpu.sync_copy(ones_v, cnt_acc.at[sidx_v.at[j]], add=True)

    plsc.subcore_barrier()

    pltpu.sync_copy(dw_acc.at[pl.ds(rbase, SR)],
                    dwt_hbm.at[c, pl.ds(rbase, SR)])

    @pl.when(c == 0)
    def _():
        pltpu.sync_copy(cnt_acc.at[pl.ds(rbase, SR)],
                        counts_hbm.at[pl.ds(rbase, SR)])


def _finalize_body(cnt_ref, dwa_ref, dwb_ref, x_ref, q_ref,
                   qst_ref, newt_ref, loss_ref, perp_ref, acc_s):
    i = pl.program_id(0)
    x = x_ref[...]
    q = q_ref[...]
    diff = q - x
    qst_ref[...] = x + diff

    part = jnp.sum(diff * diff)

    @pl.when(i == 0)
    def _():
        acc_s[0] = part

    @pl.when(i > 0)
    def _():
        acc_s[0] = acc_s[0] + part

    @pl.when(i == NB - 1)
    def _():
        loss_ref[...] = jnp.full((1, 1), COMMIT * (acc_s[0] / (N * D)),
                                 jnp.float32)

    one = jnp.float32(1.0)
    c1 = one - jnp.float32(DECAY)
    c2 = one - jnp.float32(DECAY)
    counts = cnt_ref[...]
    cs = counts * c1 / c2
    n_tot = jnp.sum(cs)
    cs_blk = cnt_ref[pl.ds(i * BN, BN), :] * c1 / c2
    csn_blk = (cs_blk + EPS) / (n_tot + K * EPS) * n_tot

    dw = jnp.concatenate([dwa_ref[0], dwb_ref[0]], axis=1)
    newt_ref[...] = (dw * c1 / c2) / csn_blk

    @pl.when(i == 0)
    def _():
        avg = counts * jnp.float32(1.0 / N)
        ent = -jnp.sum(avg * jnp.log(avg + 1e-10))
        perp_ref[...] = jnp.full((1, 1), jnp.exp(ent), jnp.float32)


def _finalize_call(counts, dwt3, x, quant):
    return pl.pallas_call(
        _finalize_body,
        grid=(NB,),
        in_specs=[
            pl.BlockSpec((K, 1), lambda i: (0, 0)),
            pl.BlockSpec((1, BN, DH), lambda i: (0, i, 0)),
            pl.BlockSpec((1, BN, DH), lambda i: (1, i, 0)),
            pl.BlockSpec((BN, D), lambda i: (i, 0)),
            pl.BlockSpec((BN, D), lambda i: (i, 0)),
        ],
        out_specs=[
            pl.BlockSpec((BN, D), lambda i: (i, 0)),
            pl.BlockSpec((BN, D), lambda i: (i, 0)),
            pl.BlockSpec((1, 1), lambda i: (0, 0)),
            pl.BlockSpec((1, 1), lambda i: (0, 0)),
        ],
        out_shape=[
            jax.ShapeDtypeStruct((N, D), jnp.float32),
            jax.ShapeDtypeStruct((K, D), jnp.float32),
            jax.ShapeDtypeStruct((1, 1), jnp.float32),
            jax.ShapeDtypeStruct((1, 1), jnp.float32),
        ],
        scratch_shapes=[pltpu.SMEM((1,), jnp.float32)],
    )(counts, dwt3, dwt3, x, quant)


def kernel(inputs, embeddings, ema_cluster_hidden, ema_dw_hidden, is_training):
    x = inputs.reshape(N, D)
    idx2d = _argmin_call(x, embeddings)
    et = embeddings.T
    idx_rows = idx2d.reshape(N // 128, 128)
    zeros = jnp.zeros((SR, DH), jnp.float32)
    zeros1 = jnp.zeros((SR,), jnp.float32)
    ones1 = jnp.ones((128,), jnp.float32)
    quant, dwt3, counts_k = _sc_kernel()(et, x, idx_rows, zeros, zeros1, ones1)
    counts = counts_k.reshape(K, 1)
    qst, newt, loss11, perp11 = _finalize_call(counts, dwt3, x, quant)

    quantized_st = qst.reshape(inputs.shape)
    encoding_indices_r = idx2d.reshape(inputs.shape[:-1])
    new_embeddings = newt.T
    return (quantized_st, loss11[0, 0], perp11[0, 0], encoding_indices_r,
            new_embeddings)
